# baseline (speedup 1.0000x reference)
import functools

import jax
import jax.numpy as jnp
from jax import lax
from jax.experimental import pallas as pl
from jax.experimental.pallas import tpu as pltpu
from jax.experimental.pallas import tpu_sc as plsc

NUM_ENTITIES = 1000000
NUM_RELATIONS = 1000
EMBED_DIM = 64
BATCH = 16384

_info = plsc.get_sparse_core_info()
NC, NS, L = _info.num_cores, _info.num_subcores, _info.num_lanes
NW = NC * NS
ROWS_PER_W = BATCH // NW
CHUNK = 128
N_CHUNKS = ROWS_PER_W // CHUNK
D_VECS = EMBED_DIM // L


def _body(e1_hbm, r_hbm, er_hbm, ei_hbm, rr_hbm, ri_hbm,
          out_r_hbm, out_i_hbm,
          idx_e, idx_r, a_buf, b_buf, c_buf, d_buf, sem):
    wid = lax.axis_index("s") * NC + lax.axis_index("c")
    base = wid * ROWS_PER_W

    def chunk_body(ci, carry):
        off = base + ci * CHUNK
        pltpu.sync_copy(e1_hbm.at[pl.ds(off, CHUNK)], idx_e)
        pltpu.sync_copy(r_hbm.at[pl.ds(off, CHUNK)], idx_r)
        cp1 = pltpu.async_copy(er_hbm.at[idx_e], a_buf, sem)
        cp2 = pltpu.async_copy(ei_hbm.at[idx_e], b_buf, sem)
        cp3 = pltpu.async_copy(rr_hbm.at[idx_r], c_buf, sem)
        cp4 = pltpu.async_copy(ri_hbm.at[idx_r], d_buf, sem)
        cp1.wait()
        cp2.wait()
        cp3.wait()
        cp4.wait()

        def row_body(row, rcarry):
            for cc in range(D_VECS):
                sl = pl.ds(cc * L, L)
                a = a_buf[row, sl]
                b = b_buf[row, sl]
                c = c_buf[row, sl]
                d = d_buf[row, sl]
                a_buf[row, sl] = a * c - b * d
                b_buf[row, sl] = a * d + b * c
            return rcarry

        lax.fori_loop(0, CHUNK, row_body, 0)
        pltpu.sync_copy(a_buf, out_r_hbm.at[pl.ds(off, CHUNK)])
        pltpu.sync_copy(b_buf, out_i_hbm.at[pl.ds(off, CHUNK)])
        return carry

    lax.fori_loop(0, N_CHUNKS, chunk_body, 0)


@jax.jit
def kernel(e1, r, ent_real, ent_img, rel_real, rel_img):
    mesh = plsc.VectorSubcoreMesh(core_axis_name="c", subcore_axis_name="s")
    out_shape = jax.ShapeDtypeStruct((BATCH, EMBED_DIM), jnp.float32)
    fn = pl.kernel(
        _body,
        out_type=(out_shape, out_shape),
        mesh=mesh,
        scratch_types=[
            pltpu.VMEM((CHUNK,), jnp.int32),
            pltpu.VMEM((CHUNK,), jnp.int32),
            pltpu.VMEM((CHUNK, EMBED_DIM), jnp.float32),
            pltpu.VMEM((CHUNK, EMBED_DIM), jnp.float32),
            pltpu.VMEM((CHUNK, EMBED_DIM), jnp.float32),
            pltpu.VMEM((CHUNK, EMBED_DIM), jnp.float32),
            pltpu.SemaphoreType.DMA,
        ],
        compiler_params=pltpu.CompilerParams(use_tc_tiling_on_sc=False),
    )
    return fn(e1, r, ent_real, ent_img, rel_real, rel_img)


# speedup vs baseline: 2.2176x; 2.2176x over previous
import jax
import jax.numpy as jnp
from jax import lax
from jax.experimental import pallas as pl
from jax.experimental.pallas import tpu as pltpu
from jax.experimental.pallas import tpu_sc as plsc

NUM_ENTITIES = 1000000
NUM_RELATIONS = 1000
EMBED_DIM = 64
BATCH = 16384

_info = plsc.get_sparse_core_info()
NC, NS, L = _info.num_cores, _info.num_subcores, _info.num_lanes
NW = NC * NS
RPW = BATCH // NW
G = 16
N_GROUPS = RPW // G
OUT_ROWS = 128
GROUPS_PER_OUT = OUT_ROWS // G
D_VECS = EMBED_DIM // L


def _body(e1_hbm, r_hbm, er3, ei3, rr3, ri3, out_r, out_i,
          eidx_v, ridx_v, a_v, b_v, c_v, d_v, or_v, oi_v, sem):
    wid = lax.axis_index("s") * NC + lax.axis_index("c")
    base = wid * RPW
    pltpu.sync_copy(e1_hbm.at[pl.ds(base, RPW)], eidx_v)
    pltpu.sync_copy(r_hbm.at[pl.ds(base, RPW)], ridx_v)

    def grp_body(g, carry):
        e_vec = eidx_v[pl.ds(g * G, G)]
        r_vec = ridx_v[pl.ds(g * G, G)]
        copies = []
        for j in range(G):
            pe = e_vec[j] >> 3
            se = e_vec[j] & 7
            pr = r_vec[j] >> 3
            sr = r_vec[j] & 7
            copies.append(pltpu.async_copy(er3.at[pe, se], a_v.at[j], sem))
            copies.append(pltpu.async_copy(ei3.at[pe, se], b_v.at[j], sem))
            copies.append(pltpu.async_copy(rr3.at[pr, sr], c_v.at[j], sem))
            copies.append(pltpu.async_copy(ri3.at[pr, sr], d_v.at[j], sem))
        for cp in copies:
            cp.wait()
        row0 = (g % GROUPS_PER_OUT) * G
        for j in range(G):
            for c in range(D_VECS):
                sl = pl.ds(c * L, L)
                a = a_v[j, sl]
                b = b_v[j, sl]
                cc = c_v[j, sl]
                d = d_v[j, sl]
                or_v[row0 + j, sl] = a * cc - b * d
                oi_v[row0 + j, sl] = a * d + b * cc

        @pl.when(g % GROUPS_PER_OUT == GROUPS_PER_OUT - 1)
        def _():
            off = base + (g // GROUPS_PER_OUT) * OUT_ROWS
            pltpu.sync_copy(or_v, out_r.at[pl.ds(off, OUT_ROWS)])
            pltpu.sync_copy(oi_v, out_i.at[pl.ds(off, OUT_ROWS)])

        return carry

    lax.fori_loop(0, N_GROUPS, grp_body, 0)


@jax.jit
def kernel(e1, r, ent_real, ent_img, rel_real, rel_img):
    er3 = ent_real.reshape(NUM_ENTITIES // 8, 8, EMBED_DIM)
    ei3 = ent_img.reshape(NUM_ENTITIES // 8, 8, EMBED_DIM)
    rr3 = rel_real.reshape(NUM_RELATIONS // 8, 8, EMBED_DIM)
    ri3 = rel_img.reshape(NUM_RELATIONS // 8, 8, EMBED_DIM)
    mesh = plsc.VectorSubcoreMesh(core_axis_name="c", subcore_axis_name="s")
    out_shape = jax.ShapeDtypeStruct((BATCH, EMBED_DIM), jnp.float32)
    fn = pl.kernel(
        _body,
        out_type=(out_shape, out_shape),
        mesh=mesh,
        scratch_types=[
            pltpu.VMEM((RPW,), jnp.int32),
            pltpu.VMEM((RPW,), jnp.int32),
            pltpu.VMEM((G, EMBED_DIM), jnp.float32),
            pltpu.VMEM((G, EMBED_DIM), jnp.float32),
            pltpu.VMEM((G, EMBED_DIM), jnp.float32),
            pltpu.VMEM((G, EMBED_DIM), jnp.float32),
            pltpu.VMEM((OUT_ROWS, EMBED_DIM), jnp.float32),
            pltpu.VMEM((OUT_ROWS, EMBED_DIM), jnp.float32),
            pltpu.SemaphoreType.DMA,
        ],
        compiler_params=pltpu.CompilerParams(
            use_tc_tiling_on_sc=True, needs_layout_passes=False),
    )
    return fn(e1, r, er3, ei3, rr3, ri3)
